# baseline (speedup 1.0000x reference)
import functools

import jax
import jax.numpy as jnp
from jax import lax
from jax.experimental import pallas as pl
from jax.experimental.pallas import tpu as pltpu
from jax.experimental.pallas import tpu_sc as plsc

S = 200
B = 4096
D = 32
L = 16
CHUNK = 128
NCH = B // CHUNK
NBUF = 2
UNROLL = 4
NEG = -1e30


def _fused_sc(emb, x):
    info = plsc.get_sparse_core_info()
    nw = info.num_cores * info.num_subcores
    base_seg, extra = S // nw, S % nw
    mesh = plsc.VectorSubcoreMesh(core_axis_name="c", subcore_axis_name="s")

    @functools.partial(
        pl.kernel,
        mesh=mesh,
        compiler_params=pltpu.CompilerParams(use_tc_tiling_on_sc=False),
        out_type=jax.ShapeDtypeStruct((S, D), jnp.float32),
        scratch_types=[
            pltpu.VMEM((NCH, CHUNK), jnp.int32),
            pltpu.VMEM((NBUF, CHUNK, D), jnp.float32),
            pltpu.VMEM((B, L), jnp.uint32),
            pltpu.VMEM((4 * NCH, L), jnp.float32),
            pltpu.VMEM((D,), jnp.float32),
            pltpu.SemaphoreType.DMA,
            pltpu.SemaphoreType.DMA,
            pltpu.SemaphoreType.DMA,
        ],
    )
    def k(table, idx_hbm, out_hbm, idx2, ring, ebf, stats, rowbuf,
          sem0, sem1, semi):
        sems = (sem0, sem1)
        wid = lax.axis_index("s") * info.num_cores + lax.axis_index("c")
        nseg = jnp.where(wid < extra, base_seg + 1, base_seg)
        seg0 = base_seg * wid + jnp.minimum(wid, extra)

        def fire(c, p):
            pltpu.async_copy(table.at[idx2.at[c]], ring.at[p], sems[p])

        def drain(p):
            pltpu.make_async_copy(
                table.at[pl.ds(0, CHUNK)], ring.at[p], sems[p]
            ).wait()

        neg_inf = jnp.full((L,), -3.4e38, jnp.float32)
        zeros = jnp.zeros((L,), jnp.float32)
        himask = jnp.full((L,), 0xFFFF0000, jnp.uint32)

        def bf16_pack(lo, hi):
            lo_u = lax.bitcast_convert_type(lo, jnp.uint32)
            hi_u = lax.bitcast_convert_type(hi, jnp.uint32)
            return (lo_u >> 16) | (hi_u & himask)

        def bf16_unpack(u):
            lo = lax.bitcast_convert_type(u << 16, jnp.float32)
            hi = lax.bitcast_convert_type(u & himask, jnp.float32)
            return lo, hi

        def do_segment(s):
            def idx_fetch(c, carry):
                pltpu.async_copy(
                    idx_hbm.at[s, pl.ds(c * CHUNK, CHUNK)], idx2.at[c], semi)
                return carry

            lax.fori_loop(0, NCH, idx_fetch, 0)

            def idx_wait(c, carry):
                pltpu.make_async_copy(
                    idx_hbm.at[s, pl.ds(0, CHUNK)], idx2.at[0], semi).wait()
                return carry

            lax.fori_loop(0, NCH, idx_wait, 0)
            for p in range(NBUF):
                fire(p, p)

            def pair(j, carry):
                for p in range(NBUF):
                    c = NBUF * j + p
                    drain(p)
                    rbuf = ring.at[p]

                    def row_max(r4, mm, _rbuf=rbuf):
                        ml, mh = mm
                        for u in range(UNROLL):
                            r = UNROLL * r4 + u
                            ml = jnp.maximum(ml, _rbuf[r, pl.ds(0, L)])
                            mh = jnp.maximum(mh, _rbuf[r, pl.ds(L, L)])
                        return ml, mh

                    m_lo, m_hi = lax.fori_loop(
                        0, CHUNK // UNROLL, row_max, (neg_inf, neg_inf))

                    def row_exp(r4, ss, _c=c, _rbuf=rbuf,
                                _ml=m_lo, _mh=m_hi):
                        sl, sh = ss
                        for u in range(UNROLL):
                            r = UNROLL * r4 + u
                            lo = _rbuf[r, pl.ds(0, L)]
                            hi = _rbuf[r, pl.ds(L, L)]
                            elo = jnp.exp(
                                jnp.where(lo == 0.0, NEG, lo) - _ml)
                            ehi = jnp.exp(
                                jnp.where(hi == 0.0, NEG, hi) - _mh)
                            sl = sl + elo
                            sh = sh + ehi
                            ebf[_c * CHUNK + r] = bf16_pack(elo, ehi)
                        return sl, sh

                    s_lo, s_hi = lax.fori_loop(
                        0, CHUNK // UNROLL, row_exp, (zeros, zeros))
                    stats[4 * c + 0] = m_lo
                    stats[4 * c + 1] = m_hi
                    stats[4 * c + 2] = s_lo
                    stats[4 * c + 3] = s_hi

                    @pl.when(c + NBUF < NCH)
                    def _(_c=c, _p=p):
                        fire(_c + NBUF, _p)

                return carry

            lax.fori_loop(0, NCH // NBUF, pair, 0)

            def mmax(c, mm):
                ml, mh = mm
                return (jnp.maximum(ml, stats[4 * c + 0]),
                        jnp.maximum(mh, stats[4 * c + 1]))

            g_lo, g_hi = lax.fori_loop(0, NCH, mmax, (neg_inf, neg_inf))

            def msum(c, ss):
                sl, sh = ss
                sl = sl + stats[4 * c + 2] * jnp.exp(stats[4 * c + 0] - g_lo)
                sh = sh + stats[4 * c + 3] * jnp.exp(stats[4 * c + 1] - g_hi)
                return sl, sh

            d_lo, d_hi = lax.fori_loop(0, NCH, msum, (zeros, zeros))
            inv_lo = 1.0 / d_lo
            inv_hi = 1.0 / d_hi

            def chunk_c(c, pp):
                f_lo = jnp.exp(stats[4 * c + 0] - g_lo) * inv_lo
                f_hi = jnp.exp(stats[4 * c + 1] - g_hi) * inv_hi

                def row_c(r4, qq, _c=c, _fl=f_lo, _fh=f_hi):
                    ql, qh = qq
                    for u in range(UNROLL):
                        r = UNROLL * r4 + u
                        elo, ehi = bf16_unpack(ebf[_c * CHUNK + r])
                        ql = ql + elo * _fl
                        qh = qh + ehi * _fh
                    return ql, qh

                return lax.fori_loop(0, CHUNK // UNROLL, row_c, pp)

            p_lo, p_hi = lax.fori_loop(0, NCH, chunk_c, (zeros, zeros))
            rowbuf[pl.ds(0, L)] = p_lo * (1.0 / B)
            rowbuf[pl.ds(L, L)] = p_hi * (1.0 / B)
            pltpu.sync_copy(rowbuf, out_hbm.at[s])

        for ki in range(base_seg + 1):
            @pl.when(ki < nseg)
            def _(_ki=ki):
                do_segment(seg0 + _ki)

    return k(emb, x)


def _linear_tc(pooled, wt, b2):
    def body(p_ref, w_ref, b_ref, o_ref):
        o_ref[...] = (
            jnp.dot(p_ref[...], w_ref[...], preferred_element_type=jnp.float32)
            + b_ref[...]
        )

    return pl.pallas_call(
        body,
        out_shape=jax.ShapeDtypeStruct((S, 2), jnp.float32),
    )(pooled, wt, b2)


def kernel(x, emb, W, b):
    pooled = _fused_sc(emb, x.astype(jnp.int32))
    return _linear_tc(pooled, W.T, b.reshape(1, 2))


# speedup vs baseline: 1.2018x; 1.2018x over previous
import functools

import jax
import jax.numpy as jnp
from jax import lax
from jax.experimental import pallas as pl
from jax.experimental.pallas import tpu as pltpu
from jax.experimental.pallas import tpu_sc as plsc

S = 200
B = 4096
D = 32
L = 16
CHUNK = 128
NCH = B // CHUNK
NBUF = 2
UNROLL = 4
NEG = -1e30


def _fused_sc(emb, x):
    info = plsc.get_sparse_core_info()
    nw = info.num_cores * info.num_subcores
    base_seg, extra = S // nw, S % nw
    mesh = plsc.VectorSubcoreMesh(core_axis_name="c", subcore_axis_name="s")

    @functools.partial(
        pl.kernel,
        mesh=mesh,
        compiler_params=pltpu.CompilerParams(use_tc_tiling_on_sc=False),
        out_type=jax.ShapeDtypeStruct((S, D), jnp.float32),
        scratch_types=[
            pltpu.VMEM((NCH, CHUNK), jnp.int32),
            pltpu.VMEM((NBUF, CHUNK, D), jnp.float32),
            pltpu.VMEM((B, L), jnp.uint32),
            pltpu.VMEM((D,), jnp.float32),
            pltpu.SemaphoreType.DMA,
            pltpu.SemaphoreType.DMA,
            pltpu.SemaphoreType.DMA,
        ],
    )
    def k(table, idx_hbm, out_hbm, idx2, ring, ebf, rowbuf,
          sem0, sem1, semi):
        sems = (sem0, sem1)
        wid = lax.axis_index("s") * info.num_cores + lax.axis_index("c")
        nseg = jnp.where(wid < extra, base_seg + 1, base_seg)
        seg0 = base_seg * wid + jnp.minimum(wid, extra)

        def fire(c, p):
            pltpu.async_copy(table.at[idx2.at[c]], ring.at[p], sems[p])

        def drain(p):
            pltpu.make_async_copy(
                table.at[pl.ds(0, CHUNK)], ring.at[p], sems[p]
            ).wait()

        neg_inf = jnp.full((L,), -3.4e38, jnp.float32)
        zeros = jnp.zeros((L,), jnp.float32)
        himask = jnp.full((L,), 0xFFFF0000, jnp.uint32)

        def bf16_pack(lo, hi):
            lo_u = lax.bitcast_convert_type(lo, jnp.uint32)
            hi_u = lax.bitcast_convert_type(hi, jnp.uint32)
            return (lo_u >> 16) | (hi_u & himask)

        def bf16_unpack(u):
            lo = lax.bitcast_convert_type(u << 16, jnp.float32)
            hi = lax.bitcast_convert_type(u & himask, jnp.float32)
            return lo, hi

        def do_segment(s):
            def idx_fetch(c, carry):
                pltpu.async_copy(
                    idx_hbm.at[s, pl.ds(c * CHUNK, CHUNK)], idx2.at[c], semi)
                return carry

            lax.fori_loop(0, NCH, idx_fetch, 0)

            def idx_wait(c, carry):
                pltpu.make_async_copy(
                    idx_hbm.at[s, pl.ds(0, CHUNK)], idx2.at[0], semi).wait()
                return carry

            lax.fori_loop(0, NCH, idx_wait, 0)
            for p in range(NBUF):
                fire(p, p)

            def pair(j, m):
                for p in range(NBUF):
                    c = NBUF * j + p
                    drain(p)
                    rbuf = ring.at[p]

                    def row_a(r4, mm, _c=c, _rbuf=rbuf):
                        mls, mhs = mm
                        mls, mhs = list(mls), list(mhs)
                        for u in range(UNROLL):
                            r = UNROLL * r4 + u
                            lo = _rbuf[r, pl.ds(0, L)]
                            hi = _rbuf[r, pl.ds(L, L)]
                            mls[u] = jnp.maximum(mls[u], lo)
                            mhs[u] = jnp.maximum(mhs[u], hi)
                            mlo = jnp.where(lo == 0.0, NEG, lo)
                            mhi = jnp.where(hi == 0.0, NEG, hi)
                            ebf[_c * CHUNK + r] = bf16_pack(mlo, mhi)
                        return tuple(mls), tuple(mhs)

                    m = lax.fori_loop(0, CHUNK // UNROLL, row_a, m)

                    @pl.when(c + NBUF < NCH)
                    def _(_c=c, _p=p):
                        fire(_c + NBUF, _p)

                return m

            m0 = ((neg_inf,) * UNROLL, (neg_inf,) * UNROLL)
            mls, mhs = lax.fori_loop(0, NCH // NBUF, pair, m0)
            m_lo, m_hi = mls[0], mhs[0]
            for u in range(1, UNROLL):
                m_lo = jnp.maximum(m_lo, mls[u])
                m_hi = jnp.maximum(m_hi, mhs[u])

            def row_b(r4, ss, _ml=m_lo, _mh=m_hi):
                sls, shs = ss
                sls, shs = list(sls), list(shs)
                for u in range(UNROLL):
                    r = UNROLL * r4 + u
                    a, bb = bf16_unpack(ebf[r])
                    elo = jnp.exp(a - _ml)
                    ehi = jnp.exp(bb - _mh)
                    sls[u] = sls[u] + elo
                    shs[u] = shs[u] + ehi
                    ebf[r] = bf16_pack(elo, ehi)
                return tuple(sls), tuple(shs)

            z0 = ((zeros,) * UNROLL, (zeros,) * UNROLL)
            sls, shs = lax.fori_loop(0, B // UNROLL, row_b, z0)
            d_lo, d_hi = sls[0], shs[0]
            for u in range(1, UNROLL):
                d_lo = d_lo + sls[u]
                d_hi = d_hi + shs[u]
            inv_lo = 1.0 / d_lo
            inv_hi = 1.0 / d_hi

            def row_c(r4, qq, _il=inv_lo, _ih=inv_hi):
                qls, qhs = qq
                qls, qhs = list(qls), list(qhs)
                for u in range(UNROLL):
                    r = UNROLL * r4 + u
                    elo, ehi = bf16_unpack(ebf[r])
                    qls[u] = qls[u] + elo * _il
                    qhs[u] = qhs[u] + ehi * _ih
                return tuple(qls), tuple(qhs)

            qls, qhs = lax.fori_loop(0, B // UNROLL, row_c, z0)
            p_lo, p_hi = qls[0], qhs[0]
            for u in range(1, UNROLL):
                p_lo = p_lo + qls[u]
                p_hi = p_hi + qhs[u]
            rowbuf[pl.ds(0, L)] = p_lo * (1.0 / B)
            rowbuf[pl.ds(L, L)] = p_hi * (1.0 / B)
            pltpu.sync_copy(rowbuf, out_hbm.at[s])

        for ki in range(base_seg + 1):
            @pl.when(ki < nseg)
            def _(_ki=ki):
                do_segment(seg0 + _ki)

    return k(emb, x)


def _linear_tc(pooled, wt, b2):
    def body(p_ref, w_ref, b_ref, o_ref):
        o_ref[...] = (
            jnp.dot(p_ref[...], w_ref[...], preferred_element_type=jnp.float32)
            + b_ref[...]
        )

    return pl.pallas_call(
        body,
        out_shape=jax.ShapeDtypeStruct((S, 2), jnp.float32),
    )(pooled, wt, b2)


def kernel(x, emb, W, b):
    pooled = _fused_sc(emb, x.astype(jnp.int32))
    return _linear_tc(pooled, W.T, b.reshape(1, 2))


# speedup vs baseline: 1.2364x; 1.0288x over previous
import functools

import jax
import jax.numpy as jnp
from jax import lax
from jax.experimental import pallas as pl
from jax.experimental.pallas import tpu as pltpu
from jax.experimental.pallas import tpu_sc as plsc

S = 200
B = 4096
D = 32
L = 16
CHUNK = 128
NCH = B // CHUNK
NBUF = 2
UNROLL = 4
NEG = -1e30


def _fused_sc(emb, x):
    info = plsc.get_sparse_core_info()
    nw = info.num_cores * info.num_subcores
    base_seg, extra = S // nw, S % nw
    mesh = plsc.VectorSubcoreMesh(core_axis_name="c", subcore_axis_name="s")

    @functools.partial(
        pl.kernel,
        mesh=mesh,
        compiler_params=pltpu.CompilerParams(use_tc_tiling_on_sc=False),
        out_type=jax.ShapeDtypeStruct((S, D), jnp.float32),
        scratch_types=[
            pltpu.VMEM((NCH, CHUNK), jnp.int32),
            pltpu.VMEM((NBUF, CHUNK, D), jnp.float32),
            pltpu.VMEM((B, L), jnp.uint32),
            pltpu.VMEM((D,), jnp.float32),
            pltpu.SemaphoreType.DMA,
            pltpu.SemaphoreType.DMA,
            pltpu.SemaphoreType.DMA,
        ],
    )
    def k(table, idx_hbm, out_hbm, idx2, ring, ebf, rowbuf,
          sem0, sem1, semi):
        sems = (sem0, sem1)
        wid = lax.axis_index("s") * info.num_cores + lax.axis_index("c")
        nseg = jnp.where(wid < extra, base_seg + 1, base_seg)
        seg0 = base_seg * wid + jnp.minimum(wid, extra)

        def fire(c, p):
            pltpu.async_copy(table.at[idx2.at[c]], ring.at[p], sems[p])

        def drain(p):
            pltpu.make_async_copy(
                table.at[pl.ds(0, CHUNK)], ring.at[p], sems[p]
            ).wait()

        neg_inf = jnp.full((L,), -3.4e38, jnp.float32)
        zeros = jnp.zeros((L,), jnp.float32)
        himask = jnp.full((L,), 0xFFFF0000, jnp.uint32)

        def bf16_pack(lo, hi):
            lo_u = lax.bitcast_convert_type(lo, jnp.uint32)
            hi_u = lax.bitcast_convert_type(hi, jnp.uint32)
            return (lo_u >> 16) | (hi_u & himask)

        def bf16_unpack(u):
            lo = lax.bitcast_convert_type(u << 16, jnp.float32)
            hi = lax.bitcast_convert_type(u & himask, jnp.float32)
            return lo, hi

        def do_segment(s):
            def idx_fetch(c, carry):
                pltpu.async_copy(
                    idx_hbm.at[s, pl.ds(c * CHUNK, CHUNK)], idx2.at[c], semi)
                return carry

            lax.fori_loop(0, NCH, idx_fetch, 0)

            def idx_wait(c, carry):
                pltpu.make_async_copy(
                    idx_hbm.at[s, pl.ds(0, CHUNK)], idx2.at[0], semi).wait()
                return carry

            lax.fori_loop(0, NCH, idx_wait, 0)
            for p in range(NBUF):
                fire(p, p)

            def pair(j, m):
                for p in range(NBUF):
                    c = NBUF * j + p
                    drain(p)
                    rbuf = ring.at[p]

                    def row_a(r4, mm, _c=c, _rbuf=rbuf):
                        ml, mh = mm
                        for u in range(UNROLL):
                            r = UNROLL * r4 + u
                            lo = _rbuf[r, pl.ds(0, L)]
                            hi = _rbuf[r, pl.ds(L, L)]
                            ml = jnp.maximum(ml, lo)
                            mh = jnp.maximum(mh, hi)
                            mlo = jnp.where(lo == 0.0, NEG, lo)
                            mhi = jnp.where(hi == 0.0, NEG, hi)
                            ebf[_c * CHUNK + r] = bf16_pack(mlo, mhi)
                        return ml, mh

                    m = lax.fori_loop(0, CHUNK // UNROLL, row_a, m)

                    @pl.when(c + NBUF < NCH)
                    def _(_c=c, _p=p):
                        fire(_c + NBUF, _p)

                return m

            m_lo, m_hi = lax.fori_loop(
                0, NCH // NBUF, pair, (neg_inf, neg_inf))

            def row_b(r4, ss, _ml=m_lo, _mh=m_hi):
                sl, sh = ss
                for u in range(UNROLL):
                    r = UNROLL * r4 + u
                    a, bb = bf16_unpack(ebf[r])
                    sl = sl + jnp.exp(a - _ml)
                    sh = sh + jnp.exp(bb - _mh)
                return sl, sh

            d_lo, d_hi = lax.fori_loop(0, B // UNROLL, row_b, (zeros, zeros))
            inv_lo = 1.0 / d_lo
            inv_hi = 1.0 / d_hi

            def row_c(r4, qq, _ml=m_lo, _mh=m_hi, _il=inv_lo, _ih=inv_hi):
                ql, qh = qq
                for u in range(UNROLL):
                    r = UNROLL * r4 + u
                    a, bb = bf16_unpack(ebf[r])
                    ql = ql + jnp.exp(a - _ml) * _il
                    qh = qh + jnp.exp(bb - _mh) * _ih
                return ql, qh

            p_lo, p_hi = lax.fori_loop(0, B // UNROLL, row_c, (zeros, zeros))
            rowbuf[pl.ds(0, L)] = p_lo * (1.0 / B)
            rowbuf[pl.ds(L, L)] = p_hi * (1.0 / B)
            pltpu.sync_copy(rowbuf, out_hbm.at[s])

        for ki in range(base_seg + 1):
            @pl.when(ki < nseg)
            def _(_ki=ki):
                do_segment(seg0 + _ki)

    return k(emb, x)


def _linear_tc(pooled, wt, b2):
    def body(p_ref, w_ref, b_ref, o_ref):
        o_ref[...] = (
            jnp.dot(p_ref[...], w_ref[...], preferred_element_type=jnp.float32)
            + b_ref[...]
        )

    return pl.pallas_call(
        body,
        out_shape=jax.ShapeDtypeStruct((S, 2), jnp.float32),
    )(pooled, wt, b2)


def kernel(x, emb, W, b):
    pooled = _fused_sc(emb, x.astype(jnp.int32))
    return _linear_tc(pooled, W.T, b.reshape(1, 2))


# speedup vs baseline: 1.2378x; 1.0012x over previous
import functools

import jax
import jax.numpy as jnp
from jax import lax
from jax.experimental import pallas as pl
from jax.experimental.pallas import tpu as pltpu
from jax.experimental.pallas import tpu_sc as plsc

S = 200
B = 4096
D = 32
L = 16
CHUNK = 128
NCH = B // CHUNK
NBUF = 2
UNROLL = 4
NEG = -1e30


def _fused_sc(emb, x):
    info = plsc.get_sparse_core_info()
    nw = info.num_cores * info.num_subcores
    base_seg, extra = S // nw, S % nw
    mesh = plsc.VectorSubcoreMesh(core_axis_name="c", subcore_axis_name="s")

    @functools.partial(
        pl.kernel,
        mesh=mesh,
        compiler_params=pltpu.CompilerParams(use_tc_tiling_on_sc=False),
        out_type=jax.ShapeDtypeStruct((S, D), jnp.float32),
        scratch_types=[
            pltpu.VMEM((NCH, CHUNK), jnp.int32),
            pltpu.VMEM((NBUF, CHUNK, D), jnp.float32),
            pltpu.VMEM((B, L), jnp.uint32),
            pltpu.VMEM((D,), jnp.float32),
            pltpu.SemaphoreType.DMA,
            pltpu.SemaphoreType.DMA,
            pltpu.SemaphoreType.DMA,
        ],
    )
    def k(table, idx_hbm, out_hbm, idx2, ring, ebf, rowbuf,
          sem0, sem1, semi):
        sems = (sem0, sem1)
        wid = lax.axis_index("s") * info.num_cores + lax.axis_index("c")
        nseg = jnp.where(wid < extra, base_seg + 1, base_seg)
        seg0 = base_seg * wid + jnp.minimum(wid, extra)

        def fire(c, p):
            pltpu.async_copy(table.at[idx2.at[c]], ring.at[p], sems[p])

        def drain(p):
            pltpu.make_async_copy(
                table.at[pl.ds(0, CHUNK)], ring.at[p], sems[p]
            ).wait()

        neg_inf = jnp.full((L,), -3.4e38, jnp.float32)
        zeros = jnp.zeros((L,), jnp.float32)
        himask = jnp.full((L,), 0xFFFF0000, jnp.uint32)

        def bf16_pack(lo, hi):
            lo_u = lax.bitcast_convert_type(lo, jnp.uint32)
            hi_u = lax.bitcast_convert_type(hi, jnp.uint32)
            return (lo_u >> 16) | (hi_u & himask)

        def bf16_unpack(u):
            lo = lax.bitcast_convert_type(u << 16, jnp.float32)
            hi = lax.bitcast_convert_type(u & himask, jnp.float32)
            return lo, hi

        def do_segment(s):
            def idx_fetch(c, carry):
                pltpu.async_copy(
                    idx_hbm.at[s, pl.ds(c * CHUNK, CHUNK)], idx2.at[c], semi)
                return carry

            lax.fori_loop(0, NCH, idx_fetch, 0)

            def idx_wait(c, carry):
                pltpu.make_async_copy(
                    idx_hbm.at[s, pl.ds(0, CHUNK)], idx2.at[0], semi).wait()
                return carry

            lax.fori_loop(0, NCH, idx_wait, 0)
            for p in range(NBUF):
                fire(p, p)

            def pair(j, m):
                for p in range(NBUF):
                    c = NBUF * j + p
                    drain(p)
                    rbuf = ring.at[p]

                    def row_a(r4, mm, _c=c, _rbuf=rbuf):
                        ml, mh = mm
                        for u in range(UNROLL):
                            r = UNROLL * r4 + u
                            lo = _rbuf[r, pl.ds(0, L)]
                            hi = _rbuf[r, pl.ds(L, L)]
                            ml = jnp.maximum(ml, lo)
                            mh = jnp.maximum(mh, hi)
                            mlo = jnp.where(lo == 0.0, NEG, lo)
                            mhi = jnp.where(hi == 0.0, NEG, hi)
                            ebf[_c * CHUNK + r] = bf16_pack(mlo, mhi)
                        return ml, mh

                    m = lax.fori_loop(0, CHUNK // UNROLL, row_a, m)

                    @pl.when(c + NBUF < NCH)
                    def _(_c=c, _p=p):
                        fire(_c + NBUF, _p)

                return m

            m_lo, m_hi = lax.fori_loop(
                0, NCH // NBUF, pair, (neg_inf, neg_inf))

            def row_b(r4, ss, _ml=m_lo, _mh=m_hi):
                sl, sh = ss
                for u in range(UNROLL):
                    r = UNROLL * r4 + u
                    a, bb = bf16_unpack(ebf[r])
                    sl = sl + jnp.exp(a - _ml)
                    sh = sh + jnp.exp(bb - _mh)
                return sl, sh

            d_lo, d_hi = lax.fori_loop(0, B // UNROLL, row_b, (zeros, zeros))
            inv_lo = 1.0 / d_lo
            inv_hi = 1.0 / d_hi

            def row_c(r4, qq, _ml=m_lo, _mh=m_hi, _il=inv_lo, _ih=inv_hi):
                ql, qh = qq
                for u in range(UNROLL):
                    r = UNROLL * r4 + u
                    a, bb = bf16_unpack(ebf[r])
                    ql = ql + jnp.exp(a - _ml) * _il
                    qh = qh + jnp.exp(bb - _mh) * _ih
                return ql, qh

            p_lo, p_hi = lax.fori_loop(0, B // UNROLL, row_c, (zeros, zeros))
            rowbuf[pl.ds(0, L)] = p_lo * (1.0 / B)
            rowbuf[pl.ds(L, L)] = p_hi * (1.0 / B)
            pltpu.sync_copy(rowbuf, out_hbm.at[s])

        def seg_body(ki, carry):
            do_segment(seg0 + ki)
            return carry

        lax.fori_loop(0, nseg, seg_body, 0)

    return k(emb, x)


def _linear_tc(pooled, wt, b2):
    def body(p_ref, w_ref, b_ref, o_ref):
        o_ref[...] = (
            jnp.dot(p_ref[...], w_ref[...], preferred_element_type=jnp.float32)
            + b_ref[...]
        )

    return pl.pallas_call(
        body,
        out_shape=jax.ShapeDtypeStruct((S, 2), jnp.float32),
    )(pooled, wt, b2)


def kernel(x, emb, W, b):
    pooled = _fused_sc(emb, x.astype(jnp.int32))
    return _linear_tc(pooled, W.T, b.reshape(1, 2))
